# baseline (speedup 1.0000x reference)
import functools

import jax
import jax.numpy as jnp
from jax import lax
from jax.experimental import pallas as pl
from jax.experimental.pallas import tpu as pltpu
from jax.experimental.pallas import tpu_sc as plsc

_CHUNK = 128


@functools.lru_cache(maxsize=None)
def _make_gather(B: int, D: int):
    info = plsc.get_sparse_core_info()
    nw = info.num_cores * info.num_subcores
    assert B % (nw * _CHUNK) == 0
    b_per_w = B // nw
    n_chunks = b_per_w // _CHUNK
    mesh = plsc.VectorSubcoreMesh(core_axis_name="c", subcore_axis_name="s")

    @functools.partial(
        pl.kernel,
        mesh=mesh,
        out_type=jax.ShapeDtypeStruct((B, D), jnp.float32),
        compiler_params=pltpu.CompilerParams(use_tc_tiling_on_sc=False),
        scratch_types=[
            pltpu.VMEM((n_chunks, _CHUNK), jnp.int32),
            pltpu.VMEM((_CHUNK, D), jnp.float32),
            pltpu.VMEM((_CHUNK, D), jnp.float32),
            pltpu.SemaphoreType.DMA,
            pltpu.SemaphoreType.DMA,
            pltpu.SemaphoreType.DMA,
            pltpu.SemaphoreType.DMA,
        ],
    )
    def gather(table_hbm, idx_hbm, out_hbm, idx_v, buf0, buf1, g0, g1, s0, s1):
        wid = lax.axis_index("s") * info.num_cores + lax.axis_index("c")
        base = wid * b_per_w
        pltpu.sync_copy(idx_hbm.at[wid], idx_v)

        bufs = (buf0, buf1)
        gsems = (g0, g1)
        ssems = (s0, s1)

        pltpu.async_copy(table_hbm.at[idx_v.at[0]], bufs[0], gsems[0])

        assert n_chunks % 2 == 0

        def group(g, _):
            for s in range(2):
                c = g * 2 + s
                o = 1 - s
                pltpu.make_async_copy(
                    table_hbm.at[idx_v.at[0]], bufs[s], gsems[s]
                ).wait()
                pltpu.async_copy(
                    bufs[s],
                    out_hbm.at[pl.ds(base + c * _CHUNK, _CHUNK)],
                    ssems[s],
                )

                @pl.when(c + 1 < n_chunks)
                def _fire(c=c, s=s, o=o):
                    @pl.when(c >= 1)
                    def _drain():
                        pltpu.make_async_copy(
                            bufs[o],
                            out_hbm.at[pl.ds(base, _CHUNK)],
                            ssems[o],
                        ).wait()

                    pltpu.async_copy(
                        table_hbm.at[idx_v.at[c + 1]], bufs[o], gsems[o]
                    )

            return 0

        lax.fori_loop(0, n_chunks // 2, group, 0)
        for s in range(2):
            pltpu.make_async_copy(
                bufs[s], out_hbm.at[pl.ds(base, _CHUNK)], ssems[s]
            ).wait()

    return gather


@functools.lru_cache(maxsize=None)
def _make_transform(V: int, D: int, blk: int):
    n_blocks = pl.cdiv(V, 2 * blk)

    def tk(t_ref, w_ref, b2_ref, o_ref):
        dn = (((0,), (1,)), ((), ()))
        o_ref[:, :D] = lax.dot_general(
            t_ref[:, :blk], w_ref[...], dn, preferred_element_type=jnp.float32
        ) + b2_ref[:, :D]
        o_ref[:, D:] = lax.dot_general(
            t_ref[:, blk:], w_ref[...], dn, preferred_element_type=jnp.float32
        ) + b2_ref[:, D:]

    return pl.pallas_call(
        tk,
        grid=(n_blocks,),
        in_specs=[
            pl.BlockSpec((D, 2 * blk), lambda i: (0, i)),
            pl.BlockSpec((D, D), lambda i: (0, 0)),
            pl.BlockSpec((1, 2 * D), lambda i: (0, 0)),
        ],
        out_specs=pl.BlockSpec((blk, 2 * D), lambda i: (i, 0)),
        out_shape=jax.ShapeDtypeStruct((n_blocks * blk, 2 * D), jnp.float32),
    )


def kernel(x, table, W, b):
    Bt, L = x.shape
    V, D = table.shape
    B = Bt * L
    blk = 16384
    info = plsc.get_sparse_core_info()
    nw = info.num_cores * info.num_subcores
    b2 = jnp.tile(b.reshape(1, D), (1, 2))
    tw = _make_transform(V, D, blk)(table.T, W, b2)
    tableW = tw.reshape(tw.shape[0] * 2, D)
    v = x.reshape(-1).astype(jnp.int32)
    flat = (v & ~(2 * blk - 1)) + 2 * (v & (blk - 1)) + ((v // blk) & 1)
    idx = flat.reshape(nw, (B // nw) // _CHUNK, _CHUNK)
    out = _make_gather(B, D)(tableW, idx)
    return out.reshape(Bt, L, D)


# speedup vs baseline: 1.0344x; 1.0344x over previous
import functools

import jax
import jax.numpy as jnp
from jax import lax
from jax.experimental import pallas as pl
from jax.experimental.pallas import tpu as pltpu
from jax.experimental.pallas import tpu_sc as plsc

_CHUNK = 256


@functools.lru_cache(maxsize=None)
def _make_gather(B: int, D: int):
    info = plsc.get_sparse_core_info()
    nw = info.num_cores * info.num_subcores
    assert B % (nw * _CHUNK) == 0
    b_per_w = B // nw
    n_chunks = b_per_w // _CHUNK
    mesh = plsc.VectorSubcoreMesh(core_axis_name="c", subcore_axis_name="s")

    @functools.partial(
        pl.kernel,
        mesh=mesh,
        out_type=jax.ShapeDtypeStruct((B, D), jnp.float32),
        compiler_params=pltpu.CompilerParams(use_tc_tiling_on_sc=False),
        scratch_types=[
            pltpu.VMEM((n_chunks, _CHUNK), jnp.int32),
            pltpu.VMEM((_CHUNK, D), jnp.float32),
            pltpu.VMEM((_CHUNK, D), jnp.float32),
            pltpu.SemaphoreType.DMA,
            pltpu.SemaphoreType.DMA,
            pltpu.SemaphoreType.DMA,
            pltpu.SemaphoreType.DMA,
        ],
    )
    def gather(table_hbm, idx_hbm, out_hbm, idx_v, buf0, buf1, g0, g1, s0, s1):
        wid = lax.axis_index("s") * info.num_cores + lax.axis_index("c")
        base = wid * b_per_w
        pltpu.sync_copy(idx_hbm.at[wid], idx_v)

        bufs = (buf0, buf1)
        gsems = (g0, g1)
        ssems = (s0, s1)

        pltpu.async_copy(table_hbm.at[idx_v.at[0]], bufs[0], gsems[0])

        n_even = (n_chunks // 2) * 2

        def group(g, _):
            for s in range(2):
                c = g * 2 + s
                o = 1 - s
                pltpu.make_async_copy(
                    table_hbm.at[idx_v.at[0]], bufs[s], gsems[s]
                ).wait()
                pltpu.async_copy(
                    bufs[s],
                    out_hbm.at[pl.ds(base + c * _CHUNK, _CHUNK)],
                    ssems[s],
                )

                @pl.when(c + 1 < n_chunks)
                def _fire(c=c, s=s, o=o):
                    @pl.when(c >= 1)
                    def _drain():
                        pltpu.make_async_copy(
                            bufs[o],
                            out_hbm.at[pl.ds(base, _CHUNK)],
                            ssems[o],
                        ).wait()

                    pltpu.async_copy(
                        table_hbm.at[idx_v.at[c + 1]], bufs[o], gsems[o]
                    )

            return 0

        lax.fori_loop(0, n_chunks // 2, group, 0)
        if n_chunks % 2:
            c = n_even
            s = c % 2
            pltpu.make_async_copy(
                table_hbm.at[idx_v.at[0]], bufs[s], gsems[s]
            ).wait()
            pltpu.async_copy(
                bufs[s],
                out_hbm.at[pl.ds(base + c * _CHUNK, _CHUNK)],
                ssems[s],
            )
        for s in range(2):
            pltpu.make_async_copy(
                bufs[s], out_hbm.at[pl.ds(base, _CHUNK)], ssems[s]
            ).wait()

    return gather


@functools.lru_cache(maxsize=None)
def _make_transform(V: int, D: int, blk: int):
    n_blocks = pl.cdiv(V, 2 * blk)

    def tk(t_ref, w_ref, b2_ref, o_ref):
        dn = (((0,), (1,)), ((), ()))
        o_ref[:, :D] = lax.dot_general(
            t_ref[:, :blk], w_ref[...], dn, preferred_element_type=jnp.float32
        ) + b2_ref[:, :D]
        o_ref[:, D:] = lax.dot_general(
            t_ref[:, blk:], w_ref[...], dn, preferred_element_type=jnp.float32
        ) + b2_ref[:, D:]

    return pl.pallas_call(
        tk,
        grid=(n_blocks,),
        in_specs=[
            pl.BlockSpec((D, 2 * blk), lambda i: (0, i)),
            pl.BlockSpec((D, D), lambda i: (0, 0)),
            pl.BlockSpec((1, 2 * D), lambda i: (0, 0)),
        ],
        out_specs=pl.BlockSpec((blk, 2 * D), lambda i: (i, 0)),
        out_shape=jax.ShapeDtypeStruct((n_blocks * blk, 2 * D), jnp.float32),
    )


def kernel(x, table, W, b):
    Bt, L = x.shape
    V, D = table.shape
    B = Bt * L
    blk = 16384
    info = plsc.get_sparse_core_info()
    nw = info.num_cores * info.num_subcores
    b2 = jnp.tile(b.reshape(1, D), (1, 2))
    tw = _make_transform(V, D, blk)(table.T, W, b2)
    tableW = tw.reshape(tw.shape[0] * 2, D)
    v = x.reshape(-1).astype(jnp.int32)
    flat = (v & ~(2 * blk - 1)) + 2 * (v & (blk - 1)) + ((v // blk) & 1)
    idx = flat.reshape(nw, (B // nw) // _CHUNK, _CHUNK)
    out = _make_gather(B, D)(tableW, idx)
    return out.reshape(Bt, L, D)


# speedup vs baseline: 1.0491x; 1.0142x over previous
import functools

import jax
import jax.numpy as jnp
from jax import lax
from jax.experimental import pallas as pl
from jax.experimental.pallas import tpu as pltpu
from jax.experimental.pallas import tpu_sc as plsc

_CHUNK = 640


@functools.lru_cache(maxsize=None)
def _make_gather(B: int, D: int):
    info = plsc.get_sparse_core_info()
    nw = info.num_cores * info.num_subcores
    assert B % (nw * _CHUNK) == 0
    b_per_w = B // nw
    n_chunks = b_per_w // _CHUNK
    mesh = plsc.VectorSubcoreMesh(core_axis_name="c", subcore_axis_name="s")

    @functools.partial(
        pl.kernel,
        mesh=mesh,
        out_type=jax.ShapeDtypeStruct((B, D), jnp.float32),
        compiler_params=pltpu.CompilerParams(use_tc_tiling_on_sc=False),
        scratch_types=[
            pltpu.VMEM((n_chunks, _CHUNK), jnp.int32),
            pltpu.VMEM((_CHUNK, D), jnp.float32),
            pltpu.VMEM((_CHUNK, D), jnp.float32),
            pltpu.SemaphoreType.DMA,
            pltpu.SemaphoreType.DMA,
            pltpu.SemaphoreType.DMA,
            pltpu.SemaphoreType.DMA,
        ],
    )
    def gather(table_hbm, idx_hbm, out_hbm, idx_v, buf0, buf1, g0, g1, s0, s1):
        wid = lax.axis_index("s") * info.num_cores + lax.axis_index("c")
        base = wid * b_per_w
        pltpu.sync_copy(idx_hbm.at[wid], idx_v)

        bufs = (buf0, buf1)
        gsems = (g0, g1)
        ssems = (s0, s1)

        pltpu.async_copy(table_hbm.at[idx_v.at[0]], bufs[0], gsems[0])

        n_even = (n_chunks // 2) * 2

        def group(g, _):
            for s in range(2):
                c = g * 2 + s
                o = 1 - s
                pltpu.make_async_copy(
                    table_hbm.at[idx_v.at[0]], bufs[s], gsems[s]
                ).wait()
                pltpu.async_copy(
                    bufs[s],
                    out_hbm.at[pl.ds(base + c * _CHUNK, _CHUNK)],
                    ssems[s],
                )

                @pl.when(c + 1 < n_chunks)
                def _fire(c=c, s=s, o=o):
                    @pl.when(c >= 1)
                    def _drain():
                        pltpu.make_async_copy(
                            bufs[o],
                            out_hbm.at[pl.ds(base, _CHUNK)],
                            ssems[o],
                        ).wait()

                    pltpu.async_copy(
                        table_hbm.at[idx_v.at[c + 1]], bufs[o], gsems[o]
                    )

            return 0

        lax.fori_loop(0, n_chunks // 2, group, 0)
        if n_chunks % 2:
            c = n_even
            s = c % 2
            pltpu.make_async_copy(
                table_hbm.at[idx_v.at[0]], bufs[s], gsems[s]
            ).wait()
            pltpu.async_copy(
                bufs[s],
                out_hbm.at[pl.ds(base + c * _CHUNK, _CHUNK)],
                ssems[s],
            )
        for s in range(2):
            pltpu.make_async_copy(
                bufs[s], out_hbm.at[pl.ds(base, _CHUNK)], ssems[s]
            ).wait()

    return gather


@functools.lru_cache(maxsize=None)
def _make_transform(V: int, D: int, blk: int):
    n_blocks = pl.cdiv(V, 2 * blk)

    def tk(t_ref, w_ref, b2_ref, o_ref):
        dn = (((0,), (1,)), ((), ()))
        o_ref[:, :D] = lax.dot_general(
            t_ref[:, :blk], w_ref[...], dn, preferred_element_type=jnp.float32
        ) + b2_ref[:, :D]
        o_ref[:, D:] = lax.dot_general(
            t_ref[:, blk:], w_ref[...], dn, preferred_element_type=jnp.float32
        ) + b2_ref[:, D:]

    return pl.pallas_call(
        tk,
        grid=(n_blocks,),
        in_specs=[
            pl.BlockSpec((D, 2 * blk), lambda i: (0, i)),
            pl.BlockSpec((D, D), lambda i: (0, 0)),
            pl.BlockSpec((1, 2 * D), lambda i: (0, 0)),
        ],
        out_specs=pl.BlockSpec((blk, 2 * D), lambda i: (i, 0)),
        out_shape=jax.ShapeDtypeStruct((n_blocks * blk, 2 * D), jnp.float32),
    )


def kernel(x, table, W, b):
    Bt, L = x.shape
    V, D = table.shape
    B = Bt * L
    blk = 16384
    info = plsc.get_sparse_core_info()
    nw = info.num_cores * info.num_subcores
    b2 = jnp.tile(b.reshape(1, D), (1, 2))
    tw = _make_transform(V, D, blk)(table.T, W, b2)
    tableW = tw.reshape(tw.shape[0] * 2, D)
    v = x.reshape(-1).astype(jnp.int32)
    flat = (v & ~(2 * blk - 1)) + 2 * (v & (blk - 1)) + ((v // blk) & 1)
    idx = flat.reshape(nw, (B // nw) // _CHUNK, _CHUNK)
    out = _make_gather(B, D)(tableW, idx)
    return out.reshape(Bt, L, D)


# speedup vs baseline: 1.0538x; 1.0045x over previous
import functools

import jax
import jax.numpy as jnp
from jax import lax
from jax.experimental import pallas as pl
from jax.experimental.pallas import tpu as pltpu
from jax.experimental.pallas import tpu_sc as plsc

_CHUNK = 800


@functools.lru_cache(maxsize=None)
def _make_gather(B: int, D: int):
    info = plsc.get_sparse_core_info()
    nw = info.num_cores * info.num_subcores
    assert B % (nw * _CHUNK) == 0
    b_per_w = B // nw
    n_chunks = b_per_w // _CHUNK
    mesh = plsc.VectorSubcoreMesh(core_axis_name="c", subcore_axis_name="s")

    @functools.partial(
        pl.kernel,
        mesh=mesh,
        out_type=jax.ShapeDtypeStruct((B, D), jnp.float32),
        compiler_params=pltpu.CompilerParams(use_tc_tiling_on_sc=False),
        scratch_types=[
            pltpu.VMEM((n_chunks, _CHUNK), jnp.int32),
            pltpu.VMEM((_CHUNK, D), jnp.float32),
            pltpu.VMEM((_CHUNK, D), jnp.float32),
            pltpu.SemaphoreType.DMA,
            pltpu.SemaphoreType.DMA,
            pltpu.SemaphoreType.DMA,
            pltpu.SemaphoreType.DMA,
        ],
    )
    def gather(table_hbm, idx_hbm, out_hbm, idx_v, buf0, buf1, g0, g1, s0, s1):
        wid = lax.axis_index("s") * info.num_cores + lax.axis_index("c")
        base = wid * b_per_w
        pltpu.sync_copy(idx_hbm.at[wid], idx_v)

        bufs = (buf0, buf1)
        gsems = (g0, g1)
        ssems = (s0, s1)

        pltpu.async_copy(table_hbm.at[idx_v.at[0]], bufs[0], gsems[0])

        n_even = (n_chunks // 2) * 2

        def group(g, _):
            for s in range(2):
                c = g * 2 + s
                o = 1 - s
                pltpu.make_async_copy(
                    table_hbm.at[idx_v.at[0]], bufs[s], gsems[s]
                ).wait()
                pltpu.async_copy(
                    bufs[s],
                    out_hbm.at[pl.ds(base + c * _CHUNK, _CHUNK)],
                    ssems[s],
                )

                @pl.when(c + 1 < n_chunks)
                def _fire(c=c, s=s, o=o):
                    @pl.when(c >= 1)
                    def _drain():
                        pltpu.make_async_copy(
                            bufs[o],
                            out_hbm.at[pl.ds(base, _CHUNK)],
                            ssems[o],
                        ).wait()

                    pltpu.async_copy(
                        table_hbm.at[idx_v.at[c + 1]], bufs[o], gsems[o]
                    )

            return 0

        lax.fori_loop(0, n_chunks // 2, group, 0)
        if n_chunks % 2:
            c = n_even
            s = c % 2
            pltpu.make_async_copy(
                table_hbm.at[idx_v.at[0]], bufs[s], gsems[s]
            ).wait()
            pltpu.async_copy(
                bufs[s],
                out_hbm.at[pl.ds(base + c * _CHUNK, _CHUNK)],
                ssems[s],
            )
        for s in range(2):
            pltpu.make_async_copy(
                bufs[s], out_hbm.at[pl.ds(base, _CHUNK)], ssems[s]
            ).wait()

    return gather


@functools.lru_cache(maxsize=None)
def _make_transform(V: int, D: int, blk: int):
    n_blocks = pl.cdiv(V, 2 * blk)

    def tk(t_ref, w_ref, b2_ref, o_ref):
        dn = (((0,), (1,)), ((), ()))
        o_ref[:, :D] = lax.dot_general(
            t_ref[:, :blk], w_ref[...], dn, preferred_element_type=jnp.float32
        ) + b2_ref[:, :D]
        o_ref[:, D:] = lax.dot_general(
            t_ref[:, blk:], w_ref[...], dn, preferred_element_type=jnp.float32
        ) + b2_ref[:, D:]

    return pl.pallas_call(
        tk,
        grid=(n_blocks,),
        in_specs=[
            pl.BlockSpec((D, 2 * blk), lambda i: (0, i)),
            pl.BlockSpec((D, D), lambda i: (0, 0)),
            pl.BlockSpec((1, 2 * D), lambda i: (0, 0)),
        ],
        out_specs=pl.BlockSpec((blk, 2 * D), lambda i: (i, 0)),
        out_shape=jax.ShapeDtypeStruct((n_blocks * blk, 2 * D), jnp.float32),
    )


def kernel(x, table, W, b):
    Bt, L = x.shape
    V, D = table.shape
    B = Bt * L
    blk = 16384
    info = plsc.get_sparse_core_info()
    nw = info.num_cores * info.num_subcores
    b2 = jnp.tile(b.reshape(1, D), (1, 2))
    tw = _make_transform(V, D, blk)(table.T, W, b2)
    tableW = tw.reshape(tw.shape[0] * 2, D)
    v = x.reshape(-1).astype(jnp.int32)
    flat = (v & ~(2 * blk - 1)) + 2 * (v & (blk - 1)) + ((v // blk) & 1)
    idx = flat.reshape(nw, (B // nw) // _CHUNK, _CHUNK)
    out = _make_gather(B, D)(tableW, idx)
    return out.reshape(Bt, L, D)
